# Initial kernel scaffold; baseline (speedup 1.0000x reference)
import jax
import jax.numpy as jnp
from jax.experimental import pallas as pl


def kernel(x, edge_index, batch, Wa, ba, ga, bea, Wb, bb, gb, beb, fc1_W, fc1_b, fc3_W, fc3_b):
    raise NotImplementedError("write your pallas kernel here")


# speedup vs baseline: 2.6969x; 2.6969x over previous
import functools

import jax
import jax.numpy as jnp
from jax import lax
from jax.experimental import pallas as pl
from jax.experimental.pallas import tpu as pltpu
from jax.experimental.pallas import tpu_sc as plsc

N = 10000
E = 320000
D = 128
G = 64
OUT = 16

NS = 16
CB = 128
CH = 157
EP = NS * CH * CB
NA = 10240
RPT = NA // NS

_mesh = plsc.VectorSubcoreMesh(core_axis_name="c", subcore_axis_name="s",
                               num_cores=1)


@functools.partial(
    pl.kernel,
    mesh=_mesh,
    out_type=jax.ShapeDtypeStruct((NA, D), jnp.float32),
    scratch_types=[
        pltpu.VMEM((CB,), jnp.int32),
        pltpu.VMEM((CB,), jnp.int32),
        pltpu.VMEM((CB, D), jnp.float32),
        pltpu.VMEM_SHARED((NA, D), jnp.float32),
        pltpu.SemaphoreType.DMA,
    ],
)
def _sc_agg(h_hbm, src_hbm, dst_hbm, out_hbm, src_v, dst_v, rows_v, acc_sh, sem):
    s = lax.axis_index("s")

    zeros16 = jnp.zeros((16,), jnp.float32)

    def _zero_row(i, carry):
        for k in range(D // 16):
            rows_v[i, pl.ds(k * 16, 16)] = zeros16
        return carry

    lax.fori_loop(0, CB, _zero_row, 0)
    for t in range(RPT // CB):
        pltpu.sync_copy(rows_v, acc_sh.at[pl.ds(s * RPT + t * CB, CB)])
    plsc.subcore_barrier()

    def _chunk(j, carry):
        pltpu.sync_copy(src_hbm.at[s, j], src_v)
        pltpu.sync_copy(dst_hbm.at[s, j], dst_v)
        pltpu.async_copy(h_hbm.at[src_v], rows_v, sem).wait()
        pltpu.sync_copy(rows_v, acc_sh.at[dst_v], add=True)
        return carry

    lax.fori_loop(0, CH, _chunk, 0)

    plsc.subcore_barrier()
    pltpu.sync_copy(acc_sh.at[pl.ds(s * RPT, RPT)],
                    out_hbm.at[pl.ds(s * RPT, RPT)])


def _mlp_body(h_ref, agg_ref, wa_ref, ba_ref, ga_ref, bea_ref,
              wb_ref, bb_ref, gb_ref, beb_ref, batch_ref,
              h_out, pool_out):
    z = h_ref[...] + agg_ref[:N, :]

    def _lin_bn_relu(v, w_ref, b_ref, g_ref, be_ref):
        y = jnp.dot(v, w_ref[...], preferred_element_type=jnp.float32)
        y = y + b_ref[...]
        m = jnp.mean(y, axis=0, keepdims=True)
        var = jnp.mean(y * y, axis=0, keepdims=True) - m * m
        y = g_ref[...] * (y - m) * lax.rsqrt(var + 1e-5) + be_ref[...]
        return jnp.maximum(y, 0.0)

    y = _lin_bn_relu(z, wa_ref, ba_ref, ga_ref, bea_ref)
    y = _lin_bn_relu(y, wb_ref, bb_ref, gb_ref, beb_ref)
    h_out[...] = y

    seg = lax.broadcasted_iota(jnp.int32, (G, N), 0)
    onehot = jnp.where(seg == batch_ref[...], 1.0, 0.0)
    pool_out[...] = jnp.dot(onehot, y, preferred_element_type=jnp.float32)


_mlp = pl.pallas_call(
    _mlp_body,
    out_shape=(jax.ShapeDtypeStruct((N, D), jnp.float32),
               jax.ShapeDtypeStruct((G, D), jnp.float32)),
)


def _head_body(x_ref, batch_ref, p1, p2, p3, p4, p5,
               fc1w_ref, fc1b_ref, fc3w_ref, fc3b_ref, out_ref):
    seg = lax.broadcasted_iota(jnp.int32, (G, N), 0)
    onehot = jnp.where(seg == batch_ref[...], 1.0, 0.0)
    px = jnp.dot(onehot, x_ref[...], preferred_element_type=jnp.float32)
    hg = jnp.concatenate(
        [px, p1[...], p2[...], p3[...], p4[...], p5[...]], axis=1)
    r = jnp.dot(hg, fc1w_ref[...], preferred_element_type=jnp.float32)
    r = jnp.maximum(r + fc1b_ref[...], 0.0)
    o = jnp.dot(r, fc3w_ref[...], preferred_element_type=jnp.float32)
    out_ref[...] = o + fc3b_ref[...]


_head = pl.pallas_call(
    _head_body,
    out_shape=jax.ShapeDtypeStruct((G, OUT), jnp.float32),
)


def kernel(x, edge_index, batch, Wa, ba, ga, bea, Wb, bb, gb, beb,
           fc1_W, fc1_b, fc3_W, fc3_b):
    src = edge_index[0].astype(jnp.int32)
    dst = edge_index[1].astype(jnp.int32)
    pad = EP - E
    src_p = jnp.concatenate([src, jnp.zeros((pad,), jnp.int32)])
    dst_p = jnp.concatenate([dst, jnp.full((pad,), N, jnp.int32)])
    src_r = src_p.reshape(NS, CH, CB)
    dst_r = dst_p.reshape(NS, CH, CB)
    batch_r = batch.astype(jnp.int32).reshape(1, N)

    h = x
    pooled = []
    for i in range(5):
        agg = _sc_agg(h, src_r, dst_r)
        h, p = _mlp(h, agg, Wa[i],
                    ba[i].reshape(1, D), ga[i].reshape(1, D),
                    bea[i].reshape(1, D), Wb[i],
                    bb[i].reshape(1, D), gb[i].reshape(1, D),
                    beb[i].reshape(1, D), batch_r)
        pooled.append(p)

    return _head(x, batch_r, *pooled,
                 fc1_W, fc1_b.reshape(1, 6 * D), fc3_W, fc3_b.reshape(1, OUT))
